# baseline (speedup 1.0000x reference)
import functools

import jax
import jax.numpy as jnp
from jax import lax
from jax.experimental import pallas as pl
from jax.experimental.pallas import tpu as pltpu
from jax.experimental.pallas import tpu_sc as plsc

N = 10000
E = 320000
D_IN = 128
D_H = 128
D_OUT = 64

N2 = 10240
CH = 160
K = 128
EP = 16 * CH * K
ROWS_PER_TILE = N2 // 16
NBUF = 4

SELU_ALPHA = 1.6732632423543772
SELU_SCALE = 1.0507009873554805

_MESH = plsc.VectorSubcoreMesh(core_axis_name="c", subcore_axis_name="s")


@functools.partial(
    pl.kernel,
    out_type=jax.ShapeDtypeStruct((2, N2), jnp.float32),
    mesh=_MESH,
    scratch_types=[
        pltpu.VMEM((CH // 2, K), jnp.int32),
        pltpu.VMEM((K,), jnp.float32),
        pltpu.VMEM((ROWS_PER_TILE,), jnp.float32),
        pltpu.VMEM_SHARED((N2,), jnp.float32),
    ],
)
def _deg_kernel(dstp_hbm, out_hbm, dst_buf, ones_buf, zero_buf, acc_sh):
    c = lax.axis_index("c")
    s = lax.axis_index("s")
    wid = s * 2 + c
    for i in range(K // 16):
        ones_buf[pl.ds(16 * i, 16)] = jnp.ones((16,), jnp.float32)
    for i in range(ROWS_PER_TILE // 16):
        zero_buf[pl.ds(16 * i, 16)] = jnp.zeros((16,), jnp.float32)
    pltpu.sync_copy(dstp_hbm.at[wid], dst_buf)
    pltpu.sync_copy(zero_buf, acc_sh.at[pl.ds(s * ROWS_PER_TILE, ROWS_PER_TILE)])
    plsc.subcore_barrier()

    def chunk(j, carry):
        pltpu.sync_copy(ones_buf, acc_sh.at[dst_buf.at[j]], add=True)
        return carry

    lax.fori_loop(0, CH // 2, chunk, 0)
    plsc.subcore_barrier()
    pltpu.sync_copy(
        acc_sh.at[pl.ds(s * ROWS_PER_TILE, ROWS_PER_TILE)],
        out_hbm.at[c, pl.ds(s * ROWS_PER_TILE, ROWS_PER_TILE)],
    )


BL = 16
NB = CH // BL


def _make_agg(h):
    @functools.partial(
        pl.kernel,
        out_type=jax.ShapeDtypeStruct((2, N2, h), jnp.float32),
        mesh=_MESH,
        compiler_params=pltpu.CompilerParams(use_tc_tiling_on_sc=False),
        scratch_types=[
            pltpu.VMEM((BL, K), jnp.int32),
            pltpu.VMEM((BL, K), jnp.int32),
            pltpu.VMEM((BL, K), jnp.int32),
            pltpu.VMEM((BL, K), jnp.int32),
            pltpu.VMEM((NBUF, K, h), jnp.float32),
            pltpu.VMEM_SHARED((N2, h), jnp.float32),
            pltpu.VMEM_SHARED((N2, h), jnp.float32),
            pltpu.SemaphoreType.DMA((NBUF,)),
            pltpu.SemaphoreType.DMA((4,)),
        ],
    )
    def _agg(g_hbm, srcp_hbm, dstp_hbm, out_hbm,
             sblk0, sblk1, dblk0, dblk1, rows_buf, g_sh, acc_sh,
             sem_g, sem_i):
        c = lax.axis_index("c")
        s = lax.axis_index("s")
        base = s * ROWS_PER_TILE
        pltpu.sync_copy(g_hbm.at[pl.ds(c * N2 + base, ROWS_PER_TILE)],
                        g_sh.at[pl.ds(base, ROWS_PER_TILE)])
        pltpu.sync_copy(g_hbm.at[pl.ds(c * N2 + base, ROWS_PER_TILE)],
                        acc_sh.at[pl.ds(base, ROWS_PER_TILE)])
        pltpu.async_copy(srcp_hbm.at[s, pl.ds(0, BL)], sblk0, sem_i.at[0])
        pltpu.async_copy(dstp_hbm.at[s, pl.ds(0, BL)], dblk0, sem_i.at[1])
        pltpu.async_copy(srcp_hbm.at[s, pl.ds(BL, BL)], sblk1, sem_i.at[2])
        pltpu.async_copy(dstp_hbm.at[s, pl.ds(BL, BL)], dblk1, sem_i.at[3])
        plsc.subcore_barrier()

        def inner(sblk, dblk):
            for b in range(NBUF):
                pltpu.async_copy(g_sh.at[sblk.at[b]], rows_buf.at[b],
                                 sem_g.at[b])
            for jj in range(BL):
                b = jj % NBUF
                pltpu.make_async_copy(g_sh.at[sblk.at[jj]], rows_buf.at[b],
                                      sem_g.at[b]).wait()
                pltpu.sync_copy(rows_buf.at[b], acc_sh.at[dblk.at[jj]],
                                add=True)
                if jj + NBUF < BL:
                    pltpu.async_copy(g_sh.at[sblk.at[jj + NBUF]],
                                     rows_buf.at[b], sem_g.at[b])

        def super_block(p, carry):
            ob = 2 * p
            pltpu.make_async_copy(srcp_hbm.at[s, pl.ds(ob * BL, BL)],
                                  sblk0, sem_i.at[0]).wait()
            pltpu.make_async_copy(dstp_hbm.at[s, pl.ds(ob * BL, BL)],
                                  dblk0, sem_i.at[1]).wait()
            inner(sblk0, dblk0)
            pltpu.async_copy(srcp_hbm.at[s, pl.ds((ob + 2) * BL, BL)],
                             sblk0, sem_i.at[0])
            pltpu.async_copy(dstp_hbm.at[s, pl.ds((ob + 2) * BL, BL)],
                             dblk0, sem_i.at[1])

            pltpu.make_async_copy(srcp_hbm.at[s, pl.ds((ob + 1) * BL, BL)],
                                  sblk1, sem_i.at[2]).wait()
            pltpu.make_async_copy(dstp_hbm.at[s, pl.ds((ob + 1) * BL, BL)],
                                  dblk1, sem_i.at[3]).wait()
            inner(sblk1, dblk1)
            pltpu.async_copy(srcp_hbm.at[s, pl.ds((ob + 3) * BL, BL)],
                             sblk1, sem_i.at[2])
            pltpu.async_copy(dstp_hbm.at[s, pl.ds((ob + 3) * BL, BL)],
                             dblk1, sem_i.at[3])
            return carry

        lax.fori_loop(0, NB // 2, super_block, 0)
        pltpu.make_async_copy(srcp_hbm.at[s, pl.ds(NB * BL, BL)],
                              sblk0, sem_i.at[0]).wait()
        pltpu.make_async_copy(dstp_hbm.at[s, pl.ds(NB * BL, BL)],
                              dblk0, sem_i.at[1]).wait()
        pltpu.make_async_copy(srcp_hbm.at[s, pl.ds((NB + 1) * BL, BL)],
                              sblk1, sem_i.at[2]).wait()
        pltpu.make_async_copy(dstp_hbm.at[s, pl.ds((NB + 1) * BL, BL)],
                              dblk1, sem_i.at[3]).wait()
        plsc.subcore_barrier()
        pltpu.sync_copy(
            acc_sh.at[pl.ds(base, ROWS_PER_TILE)],
            out_hbm.at[c, pl.ds(base, ROWS_PER_TILE)],
        )

    return _agg


_agg64 = _make_agg(D_H // 2)
_agg32 = _make_agg(D_OUT // 2)


_R = 400
_G1 = (N // _R,)
_G2 = (2, N // _R)


def _row1(w):
    return pl.BlockSpec((_R, w), lambda i: (i, 0))


def _row2(w):
    return pl.BlockSpec((_R, w), lambda c, i: (i, 0))


def _dis(d0_ref, d1_ref):
    return lax.rsqrt(d0_ref[...] + d1_ref[...] + 1.0)


def _g1_body(x_ref, w_ref, d0_ref, d1_ref, o_ref):
    h = jnp.dot(x_ref[...], w_ref[0], preferred_element_type=jnp.float32)
    o_ref[0] = _dis(d0_ref, d1_ref) * h


def _mid_body(al_ref, ah_ref, gl_ref, gh_ref, d0_ref, d1_ref, b1_ref,
              w2_ref, o_ref):
    dis = _dis(d0_ref, d1_ref)
    a = jnp.concatenate([al_ref[0] - gl_ref[0], ah_ref[0] - gh_ref[0]], axis=1)
    t = dis * a + b1_ref[...]
    neg = SELU_ALPHA * (jnp.exp(jnp.minimum(t, 0.0)) - 1.0)
    su = SELU_SCALE * jnp.where(t > 0, t, neg)
    o_ref[0] = dis * jnp.dot(su, w2_ref[0], preferred_element_type=jnp.float32)


def _out_body(al_ref, ah_ref, gl_ref, gh_ref, d0_ref, d1_ref, b2_ref, o_ref):
    a = jnp.concatenate([al_ref[0] - gl_ref[0], ah_ref[0] - gh_ref[0]], axis=1)
    z = _dis(d0_ref, d1_ref) * a + b2_ref[...]
    m = jnp.max(z, axis=1, keepdims=True)
    e = jnp.exp(z - m)
    o_ref[...] = (z - m) - jnp.log(jnp.sum(e, axis=1, keepdims=True))


def _half_in(h):
    return [pl.BlockSpec((1, _R, h), lambda c, i: (0, i, 0)),
            pl.BlockSpec((1, _R, h), lambda c, i: (1, i, 0))]


def kernel(x, edge_index, W1, b1, W2, b2):
    src = jnp.concatenate([edge_index[0], jnp.zeros((EP - E,), jnp.int32)])
    dst = jnp.concatenate([edge_index[1], jnp.full((EP - E,), N, jnp.int32)])
    pad_blocks = jnp.zeros((16, 2 * BL, K), jnp.int32)
    srcp = jnp.concatenate([src.reshape(16, CH, K), pad_blocks], axis=1)
    dstp = jnp.concatenate([dst.reshape(16, CH, K), pad_blocks], axis=1)
    dstp32 = dst.reshape(32, CH // 2, K)
    w1s = W1.T.reshape(D_IN, 2, D_H // 2).transpose(1, 0, 2)
    w2s = W2.T.reshape(D_H, 2, D_OUT // 2).transpose(1, 0, 2)

    deg = _deg_kernel(dstp32)
    d0 = deg[0].reshape(N2, 1)
    d1 = deg[1].reshape(N2, 1)

    g1 = pl.pallas_call(
        _g1_body,
        grid=_G2,
        in_specs=[_row2(D_IN),
                  pl.BlockSpec((1, D_IN, D_H // 2), lambda c, i: (c, 0, 0)),
                  _row2(1), _row2(1)],
        out_specs=pl.BlockSpec((1, _R, D_H // 2), lambda c, i: (c, i, 0)),
        out_shape=jax.ShapeDtypeStruct((2, N2, D_H // 2), jnp.float32),
    )(x, w1s, d0, d1)

    agg1 = _agg64(g1.reshape(2 * N2, D_H // 2), srcp, dstp)

    g2 = pl.pallas_call(
        _mid_body,
        grid=_G2,
        in_specs=(_half_in(D_H // 2) + _half_in(D_H // 2)
                  + [_row2(1), _row2(1),
                     pl.BlockSpec((1, D_H), lambda c, i: (0, 0)),
                     pl.BlockSpec((1, D_H, D_OUT // 2), lambda c, i: (c, 0, 0))]),
        out_specs=pl.BlockSpec((1, _R, D_OUT // 2), lambda c, i: (c, i, 0)),
        out_shape=jax.ShapeDtypeStruct((2, N2, D_OUT // 2), jnp.float32),
    )(agg1, agg1, g1, g1, d0, d1, b1.reshape(1, D_H), w2s)

    agg2 = _agg32(g2.reshape(2 * N2, D_OUT // 2), srcp, dstp)

    out = pl.pallas_call(
        _out_body,
        grid=(N // _R,),
        in_specs=[pl.BlockSpec((1, _R, D_OUT // 2), lambda i: (0, i, 0)),
                  pl.BlockSpec((1, _R, D_OUT // 2), lambda i: (1, i, 0)),
                  pl.BlockSpec((1, _R, D_OUT // 2), lambda i: (0, i, 0)),
                  pl.BlockSpec((1, _R, D_OUT // 2), lambda i: (1, i, 0)),
                  _row1(1), _row1(1),
                  pl.BlockSpec((1, D_OUT), lambda i: (0, 0))],
        out_specs=_row1(D_OUT),
        out_shape=jax.ShapeDtypeStruct((N, D_OUT), jnp.float32),
    )(agg2, agg2, g2, g2, d0, d1, b2.reshape(1, D_OUT))

    return out


# speedup vs baseline: 1.0358x; 1.0358x over previous
import functools

import jax
import jax.numpy as jnp
from jax import lax
from jax.experimental import pallas as pl
from jax.experimental.pallas import tpu as pltpu
from jax.experimental.pallas import tpu_sc as plsc

N = 10000
E = 320000
D_IN = 128
D_H = 128
D_OUT = 64

N2 = 10240
CH = 160
K = 128
EP = 16 * CH * K
ROWS_PER_TILE = N2 // 16
NBUF = 4

SELU_ALPHA = 1.6732632423543772
SELU_SCALE = 1.0507009873554805

_MESH = plsc.VectorSubcoreMesh(core_axis_name="c", subcore_axis_name="s")


@functools.partial(
    pl.kernel,
    out_type=jax.ShapeDtypeStruct((2, N2), jnp.float32),
    mesh=_MESH,
    scratch_types=[
        pltpu.VMEM((CH // 2, K), jnp.int32),
        pltpu.VMEM((K,), jnp.float32),
        pltpu.VMEM((ROWS_PER_TILE,), jnp.float32),
        pltpu.VMEM_SHARED((N2,), jnp.float32),
    ],
)
def _deg_kernel(dstp_hbm, out_hbm, dst_buf, ones_buf, zero_buf, acc_sh):
    c = lax.axis_index("c")
    s = lax.axis_index("s")
    wid = s * 2 + c
    for i in range(K // 16):
        ones_buf[pl.ds(16 * i, 16)] = jnp.ones((16,), jnp.float32)
    for i in range(ROWS_PER_TILE // 16):
        zero_buf[pl.ds(16 * i, 16)] = jnp.zeros((16,), jnp.float32)
    pltpu.sync_copy(dstp_hbm.at[wid], dst_buf)
    pltpu.sync_copy(zero_buf, acc_sh.at[pl.ds(s * ROWS_PER_TILE, ROWS_PER_TILE)])
    plsc.subcore_barrier()

    def chunk(j, carry):
        pltpu.sync_copy(ones_buf, acc_sh.at[dst_buf.at[j]], add=True)
        return carry

    lax.fori_loop(0, CH // 2, chunk, 0)
    plsc.subcore_barrier()
    pltpu.sync_copy(
        acc_sh.at[pl.ds(s * ROWS_PER_TILE, ROWS_PER_TILE)],
        out_hbm.at[c, pl.ds(s * ROWS_PER_TILE, ROWS_PER_TILE)],
    )


BL = 16
NB = CH // BL


def _make_agg(h):
    @functools.partial(
        pl.kernel,
        out_type=jax.ShapeDtypeStruct((2, N2, h), jnp.float32),
        mesh=_MESH,
        compiler_params=pltpu.CompilerParams(use_tc_tiling_on_sc=False),
        scratch_types=[
            pltpu.VMEM((BL, K), jnp.int32),
            pltpu.VMEM((BL, K), jnp.int32),
            pltpu.VMEM((BL, K), jnp.int32),
            pltpu.VMEM((BL, K), jnp.int32),
            pltpu.VMEM((NBUF, K, h), jnp.float32),
            pltpu.VMEM_SHARED((N2, h), jnp.float32),
            pltpu.VMEM_SHARED((N2, h), jnp.float32),
            pltpu.SemaphoreType.DMA((NBUF,)),
            pltpu.SemaphoreType.DMA((4,)),
        ],
    )
    def _agg(g_hbm, srcp_hbm, dstp_hbm, out_hbm,
             sblk0, sblk1, dblk0, dblk1, rows_buf, g_sh, acc_sh,
             sem_g, sem_i):
        c = lax.axis_index("c")
        s = lax.axis_index("s")
        base = s * ROWS_PER_TILE
        pltpu.sync_copy(g_hbm.at[pl.ds(c * N2 + base, ROWS_PER_TILE)],
                        g_sh.at[pl.ds(base, ROWS_PER_TILE)])
        pltpu.sync_copy(g_hbm.at[pl.ds(c * N2 + base, ROWS_PER_TILE)],
                        acc_sh.at[pl.ds(base, ROWS_PER_TILE)])
        pltpu.async_copy(srcp_hbm.at[s, pl.ds(0, BL)], sblk0, sem_i.at[0])
        pltpu.async_copy(dstp_hbm.at[s, pl.ds(0, BL)], dblk0, sem_i.at[1])
        pltpu.async_copy(srcp_hbm.at[s, pl.ds(BL, BL)], sblk1, sem_i.at[2])
        pltpu.async_copy(dstp_hbm.at[s, pl.ds(BL, BL)], dblk1, sem_i.at[3])
        plsc.subcore_barrier()

        def inner(sblk, dblk):
            for b in range(NBUF):
                pltpu.async_copy(g_sh.at[sblk.at[b]], rows_buf.at[b],
                                 sem_g.at[b])
            for jj in range(BL):
                b = jj % NBUF
                pltpu.make_async_copy(g_sh.at[sblk.at[jj]], rows_buf.at[b],
                                      sem_g.at[b]).wait()
                pltpu.sync_copy(rows_buf.at[b], acc_sh.at[dblk.at[jj]],
                                add=True)
                if jj + NBUF < BL:
                    pltpu.async_copy(g_sh.at[sblk.at[jj + NBUF]],
                                     rows_buf.at[b], sem_g.at[b])

        def super_block(p, carry):
            ob = 2 * p
            pltpu.make_async_copy(srcp_hbm.at[s, pl.ds(ob * BL, BL)],
                                  sblk0, sem_i.at[0]).wait()
            pltpu.make_async_copy(dstp_hbm.at[s, pl.ds(ob * BL, BL)],
                                  dblk0, sem_i.at[1]).wait()
            inner(sblk0, dblk0)

            @pl.when(p < NB // 2 - 1)
            def _prefetch0():
                pltpu.async_copy(srcp_hbm.at[s, pl.ds((ob + 2) * BL, BL)],
                                 sblk0, sem_i.at[0])
                pltpu.async_copy(dstp_hbm.at[s, pl.ds((ob + 2) * BL, BL)],
                                 dblk0, sem_i.at[1])

            pltpu.make_async_copy(srcp_hbm.at[s, pl.ds((ob + 1) * BL, BL)],
                                  sblk1, sem_i.at[2]).wait()
            pltpu.make_async_copy(dstp_hbm.at[s, pl.ds((ob + 1) * BL, BL)],
                                  dblk1, sem_i.at[3]).wait()
            inner(sblk1, dblk1)

            @pl.when(p < NB // 2 - 1)
            def _prefetch1():
                pltpu.async_copy(srcp_hbm.at[s, pl.ds((ob + 3) * BL, BL)],
                                 sblk1, sem_i.at[2])
                pltpu.async_copy(dstp_hbm.at[s, pl.ds((ob + 3) * BL, BL)],
                                 dblk1, sem_i.at[3])

            return carry

        lax.fori_loop(0, NB // 2, super_block, 0)
        plsc.subcore_barrier()
        pltpu.sync_copy(
            acc_sh.at[pl.ds(base, ROWS_PER_TILE)],
            out_hbm.at[c, pl.ds(base, ROWS_PER_TILE)],
        )

    return _agg


_agg64 = _make_agg(D_H // 2)
_agg32 = _make_agg(D_OUT // 2)


_R = 512
_G1 = (N2 // _R,)
_G2 = (2, N2 // _R)


def _row1(w):
    return pl.BlockSpec((_R, w), lambda i: (i, 0))


def _row2(w):
    return pl.BlockSpec((_R, w), lambda c, i: (i, 0))


def _dis(d0_ref, d1_ref):
    return lax.rsqrt(d0_ref[...] + d1_ref[...] + 1.0)


def _g1_body(x_ref, w_ref, d0_ref, d1_ref, o_ref):
    h = jnp.dot(x_ref[...], w_ref[0], preferred_element_type=jnp.float32)
    o_ref[0] = _dis(d0_ref, d1_ref) * h


def _mid_body(al_ref, ah_ref, gl_ref, gh_ref, d0_ref, d1_ref, b1_ref,
              w2_ref, o_ref):
    dis = _dis(d0_ref, d1_ref)
    a = jnp.concatenate([al_ref[0] - gl_ref[0], ah_ref[0] - gh_ref[0]], axis=1)
    t = dis * a + b1_ref[...]
    neg = SELU_ALPHA * (jnp.exp(jnp.minimum(t, 0.0)) - 1.0)
    su = SELU_SCALE * jnp.where(t > 0, t, neg)
    o_ref[0] = dis * jnp.dot(su, w2_ref[0], preferred_element_type=jnp.float32)


def _out_body(al_ref, ah_ref, gl_ref, gh_ref, d0_ref, d1_ref, b2_ref, o_ref):
    a = jnp.concatenate([al_ref[0] - gl_ref[0], ah_ref[0] - gh_ref[0]], axis=1)
    z = _dis(d0_ref, d1_ref) * a + b2_ref[...]
    m = jnp.max(z, axis=1, keepdims=True)
    e = jnp.exp(z - m)
    o_ref[...] = (z - m) - jnp.log(jnp.sum(e, axis=1, keepdims=True))


def _half_in(h):
    return [pl.BlockSpec((1, _R, h), lambda c, i: (0, i, 0)),
            pl.BlockSpec((1, _R, h), lambda c, i: (1, i, 0))]


def kernel(x, edge_index, W1, b1, W2, b2):
    src = jnp.concatenate([edge_index[0], jnp.zeros((EP - E,), jnp.int32)])
    dst = jnp.concatenate([edge_index[1], jnp.full((EP - E,), N, jnp.int32)])
    srcp = src.reshape(16, CH, K)
    dstp = dst.reshape(16, CH, K)
    xp = jnp.pad(x, ((0, N2 - N), (0, 0)))
    dstp32 = dst.reshape(32, CH // 2, K)
    w1s = W1.T.reshape(D_IN, 2, D_H // 2).transpose(1, 0, 2)
    w2s = W2.T.reshape(D_H, 2, D_OUT // 2).transpose(1, 0, 2)

    deg = _deg_kernel(dstp32)
    d0 = deg[0].reshape(N2, 1)
    d1 = deg[1].reshape(N2, 1)

    g1 = pl.pallas_call(
        _g1_body,
        grid=_G2,
        in_specs=[_row2(D_IN),
                  pl.BlockSpec((1, D_IN, D_H // 2), lambda c, i: (c, 0, 0)),
                  _row2(1), _row2(1)],
        out_specs=pl.BlockSpec((1, _R, D_H // 2), lambda c, i: (c, i, 0)),
        out_shape=jax.ShapeDtypeStruct((2, N2, D_H // 2), jnp.float32),
    )(xp, w1s, d0, d1)

    agg1 = _agg64(g1.reshape(2 * N2, D_H // 2), srcp, dstp)

    g2 = pl.pallas_call(
        _mid_body,
        grid=_G2,
        in_specs=(_half_in(D_H // 2) + _half_in(D_H // 2)
                  + [_row2(1), _row2(1),
                     pl.BlockSpec((1, D_H), lambda c, i: (0, 0)),
                     pl.BlockSpec((1, D_H, D_OUT // 2), lambda c, i: (c, 0, 0))]),
        out_specs=pl.BlockSpec((1, _R, D_OUT // 2), lambda c, i: (c, i, 0)),
        out_shape=jax.ShapeDtypeStruct((2, N2, D_OUT // 2), jnp.float32),
    )(agg1, agg1, g1, g1, d0, d1, b1.reshape(1, D_H), w2s)

    agg2 = _agg32(g2.reshape(2 * N2, D_OUT // 2), srcp, dstp)

    out = pl.pallas_call(
        _out_body,
        grid=(N2 // _R,),
        in_specs=[pl.BlockSpec((1, _R, D_OUT // 2), lambda i: (0, i, 0)),
                  pl.BlockSpec((1, _R, D_OUT // 2), lambda i: (1, i, 0)),
                  pl.BlockSpec((1, _R, D_OUT // 2), lambda i: (0, i, 0)),
                  pl.BlockSpec((1, _R, D_OUT // 2), lambda i: (1, i, 0)),
                  _row1(1), _row1(1),
                  pl.BlockSpec((1, D_OUT), lambda i: (0, 0))],
        out_specs=_row1(D_OUT),
        out_shape=jax.ShapeDtypeStruct((N2, D_OUT), jnp.float32),
    )(agg2, agg2, g2, g2, d0, d1, b2.reshape(1, D_OUT))

    return out[:N]


# speedup vs baseline: 1.0527x; 1.0164x over previous
import functools

import jax
import jax.numpy as jnp
from jax import lax
from jax.experimental import pallas as pl
from jax.experimental.pallas import tpu as pltpu
from jax.experimental.pallas import tpu_sc as plsc

N = 10000
E = 320000
D_IN = 128
D_H = 128
D_OUT = 64

N2 = 10240
CH = 160
K = 128
EP = 16 * CH * K
ROWS_PER_TILE = N2 // 16
NBUF = 4

SELU_ALPHA = 1.6732632423543772
SELU_SCALE = 1.0507009873554805

_MESH = plsc.VectorSubcoreMesh(core_axis_name="c", subcore_axis_name="s")


@functools.partial(
    pl.kernel,
    out_type=jax.ShapeDtypeStruct((2, N2), jnp.float32),
    mesh=_MESH,
    scratch_types=[
        pltpu.VMEM((CH // 2, K), jnp.int32),
        pltpu.VMEM((K,), jnp.float32),
        pltpu.VMEM((ROWS_PER_TILE,), jnp.float32),
        pltpu.VMEM_SHARED((N2,), jnp.float32),
    ],
)
def _deg_kernel(dstp_hbm, out_hbm, dst_buf, ones_buf, zero_buf, acc_sh):
    c = lax.axis_index("c")
    s = lax.axis_index("s")
    wid = s * 2 + c
    for i in range(K // 16):
        ones_buf[pl.ds(16 * i, 16)] = jnp.ones((16,), jnp.float32)
    for i in range(ROWS_PER_TILE // 16):
        zero_buf[pl.ds(16 * i, 16)] = jnp.zeros((16,), jnp.float32)
    pltpu.sync_copy(dstp_hbm.at[wid], dst_buf)
    pltpu.sync_copy(zero_buf, acc_sh.at[pl.ds(s * ROWS_PER_TILE, ROWS_PER_TILE)])
    plsc.subcore_barrier()

    def chunk(j, carry):
        pltpu.sync_copy(ones_buf, acc_sh.at[dst_buf.at[j]], add=True)
        return carry

    lax.fori_loop(0, CH // 2, chunk, 0)
    plsc.subcore_barrier()
    pltpu.sync_copy(
        acc_sh.at[pl.ds(s * ROWS_PER_TILE, ROWS_PER_TILE)],
        out_hbm.at[c, pl.ds(s * ROWS_PER_TILE, ROWS_PER_TILE)],
    )


BL = 16
NB = CH // BL


def _make_agg(h):
    @functools.partial(
        pl.kernel,
        out_type=jax.ShapeDtypeStruct((2, N2, h), jnp.float32),
        mesh=_MESH,
        compiler_params=pltpu.CompilerParams(use_tc_tiling_on_sc=False),
        scratch_types=[
            pltpu.VMEM((BL, K), jnp.int32),
            pltpu.VMEM((BL, K), jnp.int32),
            pltpu.VMEM((BL, K), jnp.int32),
            pltpu.VMEM((BL, K), jnp.int32),
            pltpu.VMEM((NBUF, K, h), jnp.float32),
            pltpu.VMEM_SHARED((N2, h), jnp.float32),
            pltpu.VMEM_SHARED((N2, h), jnp.float32),
            pltpu.SemaphoreType.DMA((NBUF,)),
            pltpu.SemaphoreType.DMA((4,)),
        ],
    )
    def _agg(g_hbm, srcp_hbm, dstp_hbm, out_hbm,
             sblk0, sblk1, dblk0, dblk1, rows_buf, g_sh, acc_sh,
             sem_g, sem_i):
        c = lax.axis_index("c")
        s = lax.axis_index("s")
        base = s * ROWS_PER_TILE
        pltpu.sync_copy(g_hbm.at[pl.ds(c * N2 + base, ROWS_PER_TILE)],
                        g_sh.at[pl.ds(base, ROWS_PER_TILE)])
        pltpu.sync_copy(g_hbm.at[pl.ds(c * N2 + base, ROWS_PER_TILE)],
                        acc_sh.at[pl.ds(base, ROWS_PER_TILE)])
        pltpu.async_copy(srcp_hbm.at[s, pl.ds(0, BL)], sblk0, sem_i.at[0])
        pltpu.async_copy(dstp_hbm.at[s, pl.ds(0, BL)], dblk0, sem_i.at[1])
        pltpu.async_copy(srcp_hbm.at[s, pl.ds(BL, BL)], sblk1, sem_i.at[2])
        pltpu.async_copy(dstp_hbm.at[s, pl.ds(BL, BL)], dblk1, sem_i.at[3])
        plsc.subcore_barrier()

        def inner(sblk, dblk):
            for b in range(NBUF):
                pltpu.async_copy(g_sh.at[sblk.at[b]], rows_buf.at[b],
                                 sem_g.at[b])
            for jj in range(BL):
                b = jj % NBUF
                pltpu.make_async_copy(g_sh.at[sblk.at[jj]], rows_buf.at[b],
                                      sem_g.at[b]).wait()
                pltpu.sync_copy(rows_buf.at[b], acc_sh.at[dblk.at[jj]],
                                add=True)
                if jj + NBUF < BL:
                    pltpu.async_copy(g_sh.at[sblk.at[jj + NBUF]],
                                     rows_buf.at[b], sem_g.at[b])

        def super_block(p, carry):
            ob = 2 * p
            pltpu.make_async_copy(srcp_hbm.at[s, pl.ds(ob * BL, BL)],
                                  sblk0, sem_i.at[0]).wait()
            pltpu.make_async_copy(dstp_hbm.at[s, pl.ds(ob * BL, BL)],
                                  dblk0, sem_i.at[1]).wait()
            inner(sblk0, dblk0)

            @pl.when(p < NB // 2 - 1)
            def _prefetch0():
                pltpu.async_copy(srcp_hbm.at[s, pl.ds((ob + 2) * BL, BL)],
                                 sblk0, sem_i.at[0])
                pltpu.async_copy(dstp_hbm.at[s, pl.ds((ob + 2) * BL, BL)],
                                 dblk0, sem_i.at[1])

            pltpu.make_async_copy(srcp_hbm.at[s, pl.ds((ob + 1) * BL, BL)],
                                  sblk1, sem_i.at[2]).wait()
            pltpu.make_async_copy(dstp_hbm.at[s, pl.ds((ob + 1) * BL, BL)],
                                  dblk1, sem_i.at[3]).wait()
            inner(sblk1, dblk1)

            @pl.when(p < NB // 2 - 1)
            def _prefetch1():
                pltpu.async_copy(srcp_hbm.at[s, pl.ds((ob + 3) * BL, BL)],
                                 sblk1, sem_i.at[2])
                pltpu.async_copy(dstp_hbm.at[s, pl.ds((ob + 3) * BL, BL)],
                                 dblk1, sem_i.at[3])

            return carry

        lax.fori_loop(0, NB // 2, super_block, 0)
        plsc.subcore_barrier()
        pltpu.sync_copy(
            acc_sh.at[pl.ds(base, ROWS_PER_TILE)],
            out_hbm.at[c, pl.ds(base, ROWS_PER_TILE)],
        )

    return _agg


_agg64 = _make_agg(D_H // 2)
_agg32 = _make_agg(D_OUT // 2)


_R = 512
_G1 = (N2 // _R,)
_G2 = (2, N2 // _R)


def _row1(w):
    return pl.BlockSpec((_R, w), lambda i: (i, 0))


def _row2(w):
    return pl.BlockSpec((_R, w), lambda c, i: (i, 0))


def _dis(d0_ref, d1_ref):
    return lax.rsqrt(d0_ref[...] + d1_ref[...] + 1.0)


def _g1_body(x_ref, w_ref, d0_ref, d1_ref, o_ref):
    h = jnp.dot(x_ref[...], w_ref[0], preferred_element_type=jnp.float32)
    o_ref[0] = _dis(d0_ref, d1_ref) * h


def _mid_body(al_ref, ah_ref, d0_ref, d1_ref, b1_ref, w2_ref, o_ref):
    dis = _dis(d0_ref, d1_ref)
    a = jnp.concatenate([al_ref[0], ah_ref[0]], axis=1)
    t = dis * a + b1_ref[...]
    neg = SELU_ALPHA * (jnp.exp(jnp.minimum(t, 0.0)) - 1.0)
    su = SELU_SCALE * jnp.where(t > 0, t, neg)
    o_ref[0] = dis * jnp.dot(su, w2_ref[0], preferred_element_type=jnp.float32)


def _out_body(al_ref, ah_ref, d0_ref, d1_ref, b2_ref, o_ref):
    a = jnp.concatenate([al_ref[0], ah_ref[0]], axis=1)
    z = _dis(d0_ref, d1_ref) * a + b2_ref[...]
    m = jnp.max(z, axis=1, keepdims=True)
    e = jnp.exp(z - m)
    o_ref[...] = (z - m) - jnp.log(jnp.sum(e, axis=1, keepdims=True))


def _half_in(h):
    return [pl.BlockSpec((1, _R, h), lambda c, i: (0, i, 0)),
            pl.BlockSpec((1, _R, h), lambda c, i: (1, i, 0))]


def kernel(x, edge_index, W1, b1, W2, b2):
    src = jnp.concatenate([edge_index[0], jnp.zeros((EP - E,), jnp.int32)])
    dst = jnp.concatenate([edge_index[1], jnp.full((EP - E,), N, jnp.int32)])
    srcp = src.reshape(16, CH, K)
    dstp = dst.reshape(16, CH, K)
    xp = jnp.pad(x, ((0, N2 - N), (0, 0)))
    dstp32 = dst.reshape(32, CH // 2, K)
    w1s = W1.T.reshape(D_IN, 2, D_H // 2).transpose(1, 0, 2)
    w2s = W2.T.reshape(D_H, 2, D_OUT // 2).transpose(1, 0, 2)

    deg = _deg_kernel(dstp32)
    d0 = deg[0].reshape(N2, 1)
    d1 = deg[1].reshape(N2, 1)

    g1 = pl.pallas_call(
        _g1_body,
        grid=_G2,
        in_specs=[_row2(D_IN),
                  pl.BlockSpec((1, D_IN, D_H // 2), lambda c, i: (c, 0, 0)),
                  _row2(1), _row2(1)],
        out_specs=pl.BlockSpec((1, _R, D_H // 2), lambda c, i: (c, i, 0)),
        out_shape=jax.ShapeDtypeStruct((2, N2, D_H // 2), jnp.float32),
    )(xp, w1s, d0, d1)

    agg1 = _agg64(g1.reshape(2 * N2, D_H // 2), srcp, dstp)

    g2 = pl.pallas_call(
        _mid_body,
        grid=_G2,
        in_specs=(_half_in(D_H // 2)
                  + [_row2(1), _row2(1),
                     pl.BlockSpec((1, D_H), lambda c, i: (0, 0)),
                     pl.BlockSpec((1, D_H, D_OUT // 2), lambda c, i: (c, 0, 0))]),
        out_specs=pl.BlockSpec((1, _R, D_OUT // 2), lambda c, i: (c, i, 0)),
        out_shape=jax.ShapeDtypeStruct((2, N2, D_OUT // 2), jnp.float32),
    )(agg1, agg1, d0, d1, b1.reshape(1, D_H), w2s)

    agg2 = _agg32(g2.reshape(2 * N2, D_OUT // 2), srcp, dstp)

    out = pl.pallas_call(
        _out_body,
        grid=(N2 // _R,),
        in_specs=[pl.BlockSpec((1, _R, D_OUT // 2), lambda i: (0, i, 0)),
                  pl.BlockSpec((1, _R, D_OUT // 2), lambda i: (1, i, 0)),
                  _row1(1), _row1(1),
                  pl.BlockSpec((1, D_OUT), lambda i: (0, 0))],
        out_specs=_row1(D_OUT),
        out_shape=jax.ShapeDtypeStruct((N2, D_OUT), jnp.float32),
    )(agg2, agg2, d0, d1, b2.reshape(1, D_OUT))

    return out[:N]


# speedup vs baseline: 1.0671x; 1.0136x over previous
import functools

import jax
import jax.numpy as jnp
from jax import lax
from jax.experimental import pallas as pl
from jax.experimental.pallas import tpu as pltpu
from jax.experimental.pallas import tpu_sc as plsc

N = 10000
E = 320000
D_IN = 128
D_H = 128
D_OUT = 64

N2 = 10240
CH = 160
K = 128
EP = 16 * CH * K
ROWS_PER_TILE = N2 // 16
NBUF = 4

SELU_ALPHA = 1.6732632423543772
SELU_SCALE = 1.0507009873554805

_MESH = plsc.VectorSubcoreMesh(core_axis_name="c", subcore_axis_name="s")


@functools.partial(
    pl.kernel,
    out_type=jax.ShapeDtypeStruct((2, N2), jnp.float32),
    mesh=_MESH,
    scratch_types=[
        pltpu.VMEM((CH // 2, K), jnp.int32),
        pltpu.VMEM((K,), jnp.float32),
        pltpu.VMEM((ROWS_PER_TILE,), jnp.float32),
        pltpu.VMEM_SHARED((N2,), jnp.float32),
    ],
)
def _deg_kernel(dstp_hbm, out_hbm, dst_buf, ones_buf, zero_buf, acc_sh):
    c = lax.axis_index("c")
    s = lax.axis_index("s")
    wid = s * 2 + c
    for i in range(K // 16):
        ones_buf[pl.ds(16 * i, 16)] = jnp.ones((16,), jnp.float32)
    for i in range(ROWS_PER_TILE // 16):
        zero_buf[pl.ds(16 * i, 16)] = jnp.zeros((16,), jnp.float32)
    pltpu.sync_copy(dstp_hbm.at[wid], dst_buf)
    pltpu.sync_copy(zero_buf, acc_sh.at[pl.ds(s * ROWS_PER_TILE, ROWS_PER_TILE)])
    plsc.subcore_barrier()

    def chunk(j, carry):
        pltpu.sync_copy(ones_buf, acc_sh.at[dst_buf.at[j]], add=True)
        return carry

    lax.fori_loop(0, CH // 2, chunk, 0)
    plsc.subcore_barrier()
    pltpu.sync_copy(
        acc_sh.at[pl.ds(s * ROWS_PER_TILE, ROWS_PER_TILE)],
        out_hbm.at[c, pl.ds(s * ROWS_PER_TILE, ROWS_PER_TILE)],
    )


BL = 20
NB = CH // BL


def _make_agg(h):
    @functools.partial(
        pl.kernel,
        out_type=jax.ShapeDtypeStruct((2, N2, h), jnp.float32),
        mesh=_MESH,
        compiler_params=pltpu.CompilerParams(use_tc_tiling_on_sc=False),
        scratch_types=[
            pltpu.VMEM((BL, K), jnp.int32),
            pltpu.VMEM((BL, K), jnp.int32),
            pltpu.VMEM((BL, K), jnp.int32),
            pltpu.VMEM((BL, K), jnp.int32),
            pltpu.VMEM((NBUF, K, h), jnp.float32),
            pltpu.VMEM_SHARED((N2, h), jnp.float32),
            pltpu.VMEM_SHARED((N2, h), jnp.float32),
            pltpu.SemaphoreType.DMA((NBUF,)),
            pltpu.SemaphoreType.DMA((4,)),
        ],
    )
    def _agg(g_hbm, srcp_hbm, dstp_hbm, out_hbm,
             sblk0, sblk1, dblk0, dblk1, rows_buf, g_sh, acc_sh,
             sem_g, sem_i):
        c = lax.axis_index("c")
        s = lax.axis_index("s")
        base = s * ROWS_PER_TILE
        pltpu.sync_copy(g_hbm.at[pl.ds(c * N2 + base, ROWS_PER_TILE)],
                        g_sh.at[pl.ds(base, ROWS_PER_TILE)])
        pltpu.sync_copy(g_hbm.at[pl.ds(c * N2 + base, ROWS_PER_TILE)],
                        acc_sh.at[pl.ds(base, ROWS_PER_TILE)])
        pltpu.async_copy(srcp_hbm.at[s, pl.ds(0, BL)], sblk0, sem_i.at[0])
        pltpu.async_copy(dstp_hbm.at[s, pl.ds(0, BL)], dblk0, sem_i.at[1])
        pltpu.async_copy(srcp_hbm.at[s, pl.ds(BL, BL)], sblk1, sem_i.at[2])
        pltpu.async_copy(dstp_hbm.at[s, pl.ds(BL, BL)], dblk1, sem_i.at[3])
        plsc.subcore_barrier()

        def inner(sblk, dblk):
            for b in range(NBUF):
                pltpu.async_copy(g_sh.at[sblk.at[b]], rows_buf.at[b],
                                 sem_g.at[b])
            for jj in range(BL):
                b = jj % NBUF
                pltpu.make_async_copy(g_sh.at[sblk.at[jj]], rows_buf.at[b],
                                      sem_g.at[b]).wait()
                pltpu.sync_copy(rows_buf.at[b], acc_sh.at[dblk.at[jj]],
                                add=True)
                if jj + NBUF < BL:
                    pltpu.async_copy(g_sh.at[sblk.at[jj + NBUF]],
                                     rows_buf.at[b], sem_g.at[b])

        def super_block(p, carry):
            ob = 2 * p
            pltpu.make_async_copy(srcp_hbm.at[s, pl.ds(ob * BL, BL)],
                                  sblk0, sem_i.at[0]).wait()
            pltpu.make_async_copy(dstp_hbm.at[s, pl.ds(ob * BL, BL)],
                                  dblk0, sem_i.at[1]).wait()
            inner(sblk0, dblk0)

            @pl.when(p < NB // 2 - 1)
            def _prefetch0():
                pltpu.async_copy(srcp_hbm.at[s, pl.ds((ob + 2) * BL, BL)],
                                 sblk0, sem_i.at[0])
                pltpu.async_copy(dstp_hbm.at[s, pl.ds((ob + 2) * BL, BL)],
                                 dblk0, sem_i.at[1])

            pltpu.make_async_copy(srcp_hbm.at[s, pl.ds((ob + 1) * BL, BL)],
                                  sblk1, sem_i.at[2]).wait()
            pltpu.make_async_copy(dstp_hbm.at[s, pl.ds((ob + 1) * BL, BL)],
                                  dblk1, sem_i.at[3]).wait()
            inner(sblk1, dblk1)

            @pl.when(p < NB // 2 - 1)
            def _prefetch1():
                pltpu.async_copy(srcp_hbm.at[s, pl.ds((ob + 3) * BL, BL)],
                                 sblk1, sem_i.at[2])
                pltpu.async_copy(dstp_hbm.at[s, pl.ds((ob + 3) * BL, BL)],
                                 dblk1, sem_i.at[3])

            return carry

        lax.fori_loop(0, NB // 2, super_block, 0)
        plsc.subcore_barrier()
        pltpu.sync_copy(
            acc_sh.at[pl.ds(base, ROWS_PER_TILE)],
            out_hbm.at[c, pl.ds(base, ROWS_PER_TILE)],
        )

    return _agg


_agg64 = _make_agg(D_H // 2)
_agg32 = _make_agg(D_OUT // 2)


_R = 512
_G1 = (N2 // _R,)
_G2 = (2, N2 // _R)


def _row1(w):
    return pl.BlockSpec((_R, w), lambda i: (i, 0))


def _row2(w):
    return pl.BlockSpec((_R, w), lambda c, i: (i, 0))


def _dis(d0_ref, d1_ref):
    return lax.rsqrt(d0_ref[...] + d1_ref[...] + 1.0)


def _g1_body(x_ref, w_ref, d0_ref, d1_ref, o_ref):
    h = jnp.dot(x_ref[...], w_ref[0], preferred_element_type=jnp.float32)
    o_ref[0] = _dis(d0_ref, d1_ref) * h


def _mid_body(al_ref, ah_ref, d0_ref, d1_ref, b1_ref, w2_ref, o_ref):
    dis = _dis(d0_ref, d1_ref)
    a = jnp.concatenate([al_ref[0], ah_ref[0]], axis=1)
    t = dis * a + b1_ref[...]
    neg = SELU_ALPHA * (jnp.exp(jnp.minimum(t, 0.0)) - 1.0)
    su = SELU_SCALE * jnp.where(t > 0, t, neg)
    o_ref[0] = dis * jnp.dot(su, w2_ref[0], preferred_element_type=jnp.float32)


def _out_body(al_ref, ah_ref, d0_ref, d1_ref, b2_ref, o_ref):
    a = jnp.concatenate([al_ref[0], ah_ref[0]], axis=1)
    z = _dis(d0_ref, d1_ref) * a + b2_ref[...]
    m = jnp.max(z, axis=1, keepdims=True)
    e = jnp.exp(z - m)
    o_ref[...] = (z - m) - jnp.log(jnp.sum(e, axis=1, keepdims=True))


def _half_in(h):
    return [pl.BlockSpec((1, _R, h), lambda c, i: (0, i, 0)),
            pl.BlockSpec((1, _R, h), lambda c, i: (1, i, 0))]


def kernel(x, edge_index, W1, b1, W2, b2):
    src = jnp.concatenate([edge_index[0], jnp.zeros((EP - E,), jnp.int32)])
    dst = jnp.concatenate([edge_index[1], jnp.full((EP - E,), N, jnp.int32)])
    srcp = src.reshape(16, CH, K)
    dstp = dst.reshape(16, CH, K)
    xp = jnp.pad(x, ((0, N2 - N), (0, 0)))
    dstp32 = dst.reshape(32, CH // 2, K)
    w1s = W1.T.reshape(D_IN, 2, D_H // 2).transpose(1, 0, 2)
    w2s = W2.T.reshape(D_H, 2, D_OUT // 2).transpose(1, 0, 2)

    deg = _deg_kernel(dstp32)
    d0 = deg[0].reshape(N2, 1)
    d1 = deg[1].reshape(N2, 1)

    g1 = pl.pallas_call(
        _g1_body,
        grid=_G2,
        in_specs=[_row2(D_IN),
                  pl.BlockSpec((1, D_IN, D_H // 2), lambda c, i: (c, 0, 0)),
                  _row2(1), _row2(1)],
        out_specs=pl.BlockSpec((1, _R, D_H // 2), lambda c, i: (c, i, 0)),
        out_shape=jax.ShapeDtypeStruct((2, N2, D_H // 2), jnp.float32),
    )(xp, w1s, d0, d1)

    agg1 = _agg64(g1.reshape(2 * N2, D_H // 2), srcp, dstp)

    g2 = pl.pallas_call(
        _mid_body,
        grid=_G2,
        in_specs=(_half_in(D_H // 2)
                  + [_row2(1), _row2(1),
                     pl.BlockSpec((1, D_H), lambda c, i: (0, 0)),
                     pl.BlockSpec((1, D_H, D_OUT // 2), lambda c, i: (c, 0, 0))]),
        out_specs=pl.BlockSpec((1, _R, D_OUT // 2), lambda c, i: (c, i, 0)),
        out_shape=jax.ShapeDtypeStruct((2, N2, D_OUT // 2), jnp.float32),
    )(agg1, agg1, d0, d1, b1.reshape(1, D_H), w2s)

    agg2 = _agg32(g2.reshape(2 * N2, D_OUT // 2), srcp, dstp)

    out = pl.pallas_call(
        _out_body,
        grid=(N2 // _R,),
        in_specs=[pl.BlockSpec((1, _R, D_OUT // 2), lambda i: (0, i, 0)),
                  pl.BlockSpec((1, _R, D_OUT // 2), lambda i: (1, i, 0)),
                  _row1(1), _row1(1),
                  pl.BlockSpec((1, D_OUT), lambda i: (0, 0))],
        out_specs=_row1(D_OUT),
        out_shape=jax.ShapeDtypeStruct((N2, D_OUT), jnp.float32),
    )(agg2, agg2, d0, d1, b2.reshape(1, D_OUT))

    return out[:N]
